# baseline (speedup 1.0000x reference)
import functools

import jax
import jax.numpy as jnp
from jax import lax
from jax.experimental import pallas as pl
from jax.experimental.pallas import tpu as pltpu
from jax.experimental.pallas import tpu_sc as plsc

N = 10000
E = 160000
H = 128
ND = 10
ED = 9

NC = 2
NS = 16
NW = NC * NS
CH = 128

BN = 1000
BE = 1000

_f32 = jnp.float32


def _ln(y, g, b):
    mu = jnp.mean(y, axis=-1, keepdims=True)
    var = jnp.mean((y - mu) ** 2, axis=-1, keepdims=True)
    return (y - mu) / jnp.sqrt(var + 1e-5) * g + b


def _w(shape):
    return pl.BlockSpec(shape, lambda i: (0,) * len(shape))


def _enc_node_body(x, w1, b1, w2, b2, w3, b3, g, bt, wi, wj, n_out, ab_out):
    h = jnp.maximum(x[...] @ w1[...] + b1[...], 0.0)
    h = jnp.maximum(h @ w2[...] + b2[...], 0.0)
    y = h @ w3[...] + b3[...]
    n = _ln(y, g[...], bt[...])
    n_out[...] = n
    ab_out[0] = n @ wi[...]
    ab_out[1] = n @ wj[...]


def _enc_edge_body(x, w1, b1, w2, b2, w3, b3, g, bt, e_out):
    h = jnp.maximum(x[...] @ w1[...] + b1[...], 0.0)
    h = jnp.maximum(h @ w2[...] + b2[...], 0.0)
    y = h @ w3[...] + b3[...]
    e_out[...] = _ln(y, g[...], bt[...])


def _edge_body(ga, gb, e, w1c, b1, w2, b2, w3, b3, g, bt, out):
    ev = e[...]
    t = jnp.maximum(ga[...] + gb[...] + ev @ w1c[...] + b1[...], 0.0)
    h = jnp.maximum(t @ w2[...] + b2[...], 0.0)
    y = h @ w3[...] + b3[...]
    out[...] = _ln(y, g[...], bt[...]) + ev


def _node_body(nref, pe, pc, w1n, w1a, b1, w2, b2, w3, b3, g, bt, wi, wj,
               n_out, ab_out):
    n = nref[...]
    cnt = jnp.maximum((pc[0] + pc[1])[:, 0:1], 1.0)
    agg = (pe[0] + pe[1]) / cnt
    x = jnp.maximum(n @ w1n[...] + agg @ w1a[...] + b1[...], 0.0)
    h = jnp.maximum(x @ w2[...] + b2[...], 0.0)
    y = h @ w3[...] + b3[...]
    nn = _ln(y, g[...], bt[...]) + n
    n_out[...] = nn
    ab_out[0] = nn @ wi[...]
    ab_out[1] = nn @ wj[...]


def _node_final_body(nref, pe, pc, w1n, w1a, b1, w2, b2, w3, b3, g, bt,
                     wd1, bd1, wd2, bd2, wd3, bd3, out):
    n = nref[...]
    cnt = jnp.maximum((pc[0] + pc[1])[:, 0:1], 1.0)
    agg = (pe[0] + pe[1]) / cnt
    x = jnp.maximum(n @ w1n[...] + agg @ w1a[...] + b1[...], 0.0)
    h = jnp.maximum(x @ w2[...] + b2[...], 0.0)
    y = h @ w3[...] + b3[...]
    nn = _ln(y, g[...], bt[...]) + n
    d = jnp.maximum(nn @ wd1[...] + bd1[...], 0.0)
    d = jnp.maximum(d @ wd2[...] + bd2[...], 0.0)
    out[...] = d @ wd3[...] + bd3[...]


_R_S = E // CH
_NBUF = 4
_NRT = 80
_R_GP = _NRT * NW
_EP = _R_GP * CH


@functools.lru_cache(maxsize=None)
def _sc_kernels():
    mesh = plsc.VectorSubcoreMesh(core_axis_name="c", subcore_axis_name="s",
                                  num_cores=NC, num_subcores=NS)

    @functools.partial(
        pl.kernel,
        out_type=jax.ShapeDtypeStruct((_EP, H), _f32),
        mesh=mesh,
        scratch_types=[
            pltpu.VMEM((CH,), jnp.int32),
            pltpu.VMEM((CH, H), _f32),
            pltpu.SemaphoreType.DMA,
        ],
    )
    def _sc_gather(t_hbm, j_hbm, g_hbm, idx_v, buf_v, gsem):
        w = lax.axis_index("s") * NC + lax.axis_index("c")

        def body(j, carry):
            row = w + j * NW
            pltpu.sync_copy(j_hbm.at[row], idx_v)
            pltpu.async_copy(t_hbm.at[idx_v], buf_v, gsem).wait()
            pltpu.sync_copy(buf_v, g_hbm.at[pl.ds(row * CH, CH)])
            return carry

        lax.fori_loop(0, _NRT, body, 0)

    @functools.partial(
        pl.kernel,
        out_type=jax.ShapeDtypeStruct((NC, N, H), _f32),
        mesh=mesh,
        scratch_types=[
            pltpu.VMEM((CH,), jnp.int32),
            pltpu.VMEM((CH, H), _f32),
            pltpu.VMEM_SHARED((N, H), _f32),
            pltpu.SemaphoreType.DMA,
        ],
    )
    def _sc_scatter(x_hbm, d_hbm, z_hbm, p_hbm, idx_v, buf_v, acc, sem):
        c = lax.axis_index("c")
        s = lax.axis_index("s")

        @pl.when(s == 0)
        def _():
            pltpu.sync_copy(z_hbm, acc)

        plsc.subcore_barrier()

        half = _R_S // NC
        nloop = (half + NS - 1) // NS

        def body(j, carry):
            rl = s + j * NS

            @pl.when(rl < half)
            def _():
                row = c * half + rl
                pltpu.sync_copy(d_hbm.at[row], idx_v)
                pltpu.sync_copy(x_hbm.at[pl.ds(row * CH, CH)], buf_v)
                pltpu.sync_copy(buf_v, acc.at[idx_v], add=True)

            return carry

        lax.fori_loop(0, nloop, body, 0)
        plsc.subcore_barrier()
        rows = 1000

        @pl.when(s < N // rows)
        def _():
            pltpu.sync_copy(acc.at[pl.ds(s * rows, rows)],
                            p_hbm.at[c, pl.ds(s * rows, rows)])

    @functools.partial(
        pl.kernel,
        out_type=jax.ShapeDtypeStruct((NC, N, H), _f32),
        mesh=mesh,
        scratch_types=[
            pltpu.VMEM((CH,), jnp.int32),
            pltpu.VMEM((CH, H), _f32),
            pltpu.VMEM_SHARED((N, H), _f32),
            pltpu.SemaphoreType.DMA,
        ],
    )
    def _sc_count(d_hbm, ones_hbm, z_hbm, p_hbm, idx_v, buf_v, acc, sem):
        c = lax.axis_index("c")
        s = lax.axis_index("s")

        @pl.when(s == 0)
        def _():
            pltpu.sync_copy(z_hbm, acc)

        pltpu.sync_copy(ones_hbm, buf_v)
        plsc.subcore_barrier()

        half = _R_S // NC
        nloop = (half + NS - 1) // NS

        def body(j, carry):
            rl = s + j * NS

            @pl.when(rl < half)
            def _():
                row = c * half + rl
                pltpu.sync_copy(d_hbm.at[row], idx_v)
                pltpu.sync_copy(buf_v, acc.at[idx_v], add=True)

            return carry

        lax.fori_loop(0, nloop, body, 0)
        plsc.subcore_barrier()
        rows = 1000

        @pl.when(s < N // rows)
        def _():
            pltpu.sync_copy(acc.at[pl.ds(s * rows, rows)],
                            p_hbm.at[c, pl.ds(s * rows, rows)])

    return _sc_gather, _sc_scatter, _sc_count


def _gather_rows(table, j2):
    return _sc_kernels()[0](table, j2)


def _scatter_rows(x, d2, zeros_nh):
    return _sc_kernels()[1](x, d2, zeros_nh)


def _count_rows(d2, ones16, zeros16):
    return _sc_kernels()[2](d2, ones16, zeros16)


def _lin_w(p, name):
    return p[name]["w"], p[name]["b"].reshape(1, H)


def kernel(node_features, edge_features, edge_index, params):
    src = edge_index[0]
    dst = edge_index[1]
    pad = _EP - 2 * E
    j2 = jnp.concatenate(
        [src, dst + N, jnp.zeros((pad,), jnp.int32)]).reshape(_R_GP, CH)
    d2 = dst.reshape(_R_S, CH)

    zeros_nh = jnp.zeros((N, H), _f32)
    zeros16 = jnp.zeros((N, H), _f32)
    ones16 = jnp.ones((CH, H), _f32)

    ne = params["node_enc"]
    ee = params["edge_enc"]
    procs = params["procs"]
    dec = params["dec"]

    def mlp_args(p):
        w1, b1 = _lin_w(p, "l1")
        w2, b2 = _lin_w(p, "l2")
        w3, b3 = _lin_w(p, "l3")
        return (w1, b1, w2, b2, w3, b3,
                p["gamma"].reshape(1, H), p["beta"].reshape(1, H))

    ew = []
    for p in procs:
        w1 = p["edge_mlp"]["l1"]["w"]
        ew.append((w1[:H], w1[H:2 * H], w1[2 * H:]))

    enc_n_args = mlp_args(ne)
    n0, ab = pl.pallas_call(
        _enc_node_body,
        grid=(N // BN,),
        in_specs=[pl.BlockSpec((BN, ND), lambda i: (i, 0)),
                  _w((ND, H)), _w((1, H)), _w((H, H)), _w((1, H)),
                  _w((H, H)), _w((1, H)), _w((1, H)), _w((1, H)),
                  _w((H, H)), _w((H, H))],
        out_specs=[pl.BlockSpec((BN, H), lambda i: (i, 0)),
                   pl.BlockSpec((2, BN, H), lambda i: (0, i, 0))],
        out_shape=[jax.ShapeDtypeStruct((N, H), _f32),
                   jax.ShapeDtypeStruct((2, N, H), _f32)],
    )(node_features, *enc_n_args, ew[0][0], ew[0][1])

    enc_e_args = mlp_args(ee)
    e = pl.pallas_call(
        _enc_edge_body,
        grid=(E // BE,),
        in_specs=[pl.BlockSpec((BE, ED), lambda i: (i, 0)),
                  _w((ED, H)), _w((1, H)), _w((H, H)), _w((1, H)),
                  _w((H, H)), _w((1, H)), _w((1, H)), _w((1, H))],
        out_specs=pl.BlockSpec((BE, H), lambda i: (i, 0)),
        out_shape=jax.ShapeDtypeStruct((E, H), _f32),
    )(edge_features, *enc_e_args)

    pc = _count_rows(d2, ones16, zeros16)

    n = n0
    for step in range(4):
        p = procs[step]
        table = ab.reshape(2 * N, H)
        g = _gather_rows(table, j2)

        em = mlp_args(p["edge_mlp"])
        w1c = ew[step][2]
        e = pl.pallas_call(
            _edge_body,
            grid=(E // BE,),
            in_specs=[pl.BlockSpec((BE, H), lambda i: (i, 0)),
                      pl.BlockSpec((BE, H), lambda i: (E // BE + i, 0)),
                      pl.BlockSpec((BE, H), lambda i: (i, 0)),
                      _w((H, H)), _w((1, H)), _w((H, H)), _w((1, H)),
                      _w((H, H)), _w((1, H)), _w((1, H)), _w((1, H))],
            out_specs=pl.BlockSpec((BE, H), lambda i: (i, 0)),
            out_shape=jax.ShapeDtypeStruct((E, H), _f32),
        )(g, g, e, w1c, em[1], em[2], em[3], em[4], em[5], em[6], em[7])

        pe = _scatter_rows(e, d2, zeros_nh)

        nm = mlp_args(p["node_mlp"])
        w1full = p["node_mlp"]["l1"]["w"]
        w1n, w1a = w1full[:H], w1full[H:]
        common_specs = [
            pl.BlockSpec((BN, H), lambda i: (i, 0)),
            pl.BlockSpec((2, BN, H), lambda i: (0, i, 0)),
            pl.BlockSpec((2, BN, H), lambda i: (0, i, 0)),
            _w((H, H)), _w((H, H)), _w((1, H)), _w((H, H)), _w((1, H)),
            _w((H, H)), _w((1, H)), _w((1, H)), _w((1, H))]
        if step < 3:
            n, ab = pl.pallas_call(
                _node_body,
                grid=(N // BN,),
                in_specs=common_specs + [_w((H, H)), _w((H, H))],
                out_specs=[pl.BlockSpec((BN, H), lambda i: (i, 0)),
                           pl.BlockSpec((2, BN, H), lambda i: (0, i, 0))],
                out_shape=[jax.ShapeDtypeStruct((N, H), _f32),
                           jax.ShapeDtypeStruct((2, N, H), _f32)],
            )(n, pe, pc, w1n, w1a, nm[1], nm[2], nm[3], nm[4], nm[5],
              nm[6], nm[7], ew[step + 1][0], ew[step + 1][1])
        else:
            wd1, bd1 = _lin_w(dec, "l1")
            wd2, bd2 = _lin_w(dec, "l2")
            wd3 = jnp.pad(dec["l3"]["w"], ((0, 0), (0, H - 3)))
            bd3 = jnp.pad(dec["l3"]["b"], (0, H - 3)).reshape(1, H)
            out = pl.pallas_call(
                _node_final_body,
                grid=(N // BN,),
                in_specs=common_specs + [_w((H, H)), _w((1, H)),
                                         _w((H, H)), _w((1, H)),
                                         _w((H, H)), _w((1, H))],
                out_specs=pl.BlockSpec((BN, H), lambda i: (i, 0)),
                out_shape=jax.ShapeDtypeStruct((N, H), _f32),
            )(n, pe, pc, w1n, w1a, nm[1], nm[2], nm[3], nm[4], nm[5],
              nm[6], nm[7], wd1, bd1, wd2, bd2, wd3, bd3)

    return out[:, :3]


# speedup vs baseline: 1.5052x; 1.5052x over previous
import functools

import jax
import jax.numpy as jnp
from jax import lax
from jax.experimental import pallas as pl
from jax.experimental.pallas import tpu as pltpu
from jax.experimental.pallas import tpu_sc as plsc

N = 10000
E = 160000
H = 128
ND = 10
ED = 9

NC = 2
NS = 16
NW = NC * NS
CH = 128

BN = 1000
BE = 1000

_f32 = jnp.float32


def _ln(y, g, b):
    mu = jnp.mean(y, axis=-1, keepdims=True)
    var = jnp.mean((y - mu) ** 2, axis=-1, keepdims=True)
    return (y - mu) / jnp.sqrt(var + 1e-5) * g + b


def _w(shape):
    return pl.BlockSpec(shape, lambda i: (0,) * len(shape))


def _enc_node_body(x, w1, b1, w2, b2, w3, b3, g, bt, wi, wj, n_out, ab_out):
    h = jnp.maximum(x[...] @ w1[...] + b1[...], 0.0)
    h = jnp.maximum(h @ w2[...] + b2[...], 0.0)
    y = h @ w3[...] + b3[...]
    n = _ln(y, g[...], bt[...])
    n_out[...] = n
    ab_out[0] = n @ wi[...]
    ab_out[1] = n @ wj[...]


def _enc_edge_body(x, w1, b1, w2, b2, w3, b3, g, bt, e_out):
    h = jnp.maximum(x[...] @ w1[...] + b1[...], 0.0)
    h = jnp.maximum(h @ w2[...] + b2[...], 0.0)
    y = h @ w3[...] + b3[...]
    e_out[...] = _ln(y, g[...], bt[...])


def _edge_body(ga, gb, e, w1c, b1, w2, b2, w3, b3, g, bt, out):
    ev = e[...]
    t = jnp.maximum(ga[...] + gb[...] + ev @ w1c[...] + b1[...], 0.0)
    h = jnp.maximum(t @ w2[...] + b2[...], 0.0)
    y = h @ w3[...] + b3[...]
    out[...] = _ln(y, g[...], bt[...]) + ev


def _node_body(nref, pe, pc, w1n, w1a, b1, w2, b2, w3, b3, g, bt, wi, wj,
               n_out, ab_out):
    n = nref[...]
    cnt = jnp.maximum((pc[0] + pc[1])[:, 0:1], 1.0)
    agg = (pe[0] + pe[1]) / cnt
    x = jnp.maximum(n @ w1n[...] + agg @ w1a[...] + b1[...], 0.0)
    h = jnp.maximum(x @ w2[...] + b2[...], 0.0)
    y = h @ w3[...] + b3[...]
    nn = _ln(y, g[...], bt[...]) + n
    n_out[...] = nn
    ab_out[0] = nn @ wi[...]
    ab_out[1] = nn @ wj[...]


def _node_final_body(nref, pe, pc, w1n, w1a, b1, w2, b2, w3, b3, g, bt,
                     wd1, bd1, wd2, bd2, wd3, bd3, out):
    n = nref[...]
    cnt = jnp.maximum((pc[0] + pc[1])[:, 0:1], 1.0)
    agg = (pe[0] + pe[1]) / cnt
    x = jnp.maximum(n @ w1n[...] + agg @ w1a[...] + b1[...], 0.0)
    h = jnp.maximum(x @ w2[...] + b2[...], 0.0)
    y = h @ w3[...] + b3[...]
    nn = _ln(y, g[...], bt[...]) + n
    d = jnp.maximum(nn @ wd1[...] + bd1[...], 0.0)
    d = jnp.maximum(d @ wd2[...] + bd2[...], 0.0)
    out[...] = d @ wd3[...] + bd3[...]


_R_S = E // CH
_NBUF = 4
_NRT = 80
_R_GP = _NRT * NW
_EP = _R_GP * CH


@functools.lru_cache(maxsize=None)
def _sc_kernels():
    mesh = plsc.VectorSubcoreMesh(core_axis_name="c", subcore_axis_name="s",
                                  num_cores=NC, num_subcores=NS)

    @functools.partial(
        pl.kernel,
        out_type=jax.ShapeDtypeStruct((2 * E, H), _f32),
        mesh=mesh,
        scratch_types=[
            pltpu.VMEM((CH,), jnp.int32),
            pltpu.VMEM((CH, H), _f32),
            pltpu.SemaphoreType.DMA,
        ],
    )
    def _sc_gather(t_hbm, j_hbm, g_hbm, idx_v, buf_v, gsem):
        w = lax.axis_index("s") * NC + lax.axis_index("c")
        nrow = 2 * E // CH

        def body(j, carry):
            row = w + j * NW

            @pl.when(row < nrow)
            def _():
                pltpu.sync_copy(j_hbm.at[row], idx_v)
                pltpu.async_copy(t_hbm.at[idx_v], buf_v, gsem).wait()
                pltpu.sync_copy(buf_v, g_hbm.at[pl.ds(row * CH, CH)])

            return carry

        lax.fori_loop(0, _NRT, body, 0)

    @functools.partial(
        pl.kernel,
        out_type=jax.ShapeDtypeStruct((NC, N, H), _f32),
        mesh=mesh,
        scratch_types=[
            pltpu.VMEM((CH,), jnp.int32),
            pltpu.VMEM((CH, H), _f32),
            pltpu.VMEM_SHARED((N, H), _f32),
            pltpu.SemaphoreType.DMA,
        ],
    )
    def _sc_scatter(x_hbm, d_hbm, z_hbm, p_hbm, idx_v, buf_v, acc, sem):
        c = lax.axis_index("c")
        s = lax.axis_index("s")

        @pl.when(s == 0)
        def _():
            pltpu.sync_copy(z_hbm, acc)

        plsc.subcore_barrier()

        half = _R_S // NC
        nloop = (half + NS - 1) // NS

        def body(j, carry):
            rl = s + j * NS

            @pl.when(rl < half)
            def _():
                row = c * half + rl
                pltpu.sync_copy(d_hbm.at[row], idx_v)
                pltpu.sync_copy(x_hbm.at[pl.ds(row * CH, CH)], buf_v)
                pltpu.sync_copy(buf_v, acc.at[idx_v], add=True)

            return carry

        lax.fori_loop(0, nloop, body, 0)
        plsc.subcore_barrier()
        rows = 1000

        @pl.when(s < N // rows)
        def _():
            pltpu.sync_copy(acc.at[pl.ds(s * rows, rows)],
                            p_hbm.at[c, pl.ds(s * rows, rows)])

    @functools.partial(
        pl.kernel,
        out_type=jax.ShapeDtypeStruct((NC, N, H), _f32),
        mesh=mesh,
        scratch_types=[
            pltpu.VMEM((CH,), jnp.int32),
            pltpu.VMEM((CH, H), _f32),
            pltpu.VMEM_SHARED((N, H), _f32),
            pltpu.SemaphoreType.DMA,
        ],
    )
    def _sc_count(d_hbm, ones_hbm, z_hbm, p_hbm, idx_v, buf_v, acc, sem):
        c = lax.axis_index("c")
        s = lax.axis_index("s")

        @pl.when(s == 0)
        def _():
            pltpu.sync_copy(z_hbm, acc)

        pltpu.sync_copy(ones_hbm, buf_v)
        plsc.subcore_barrier()

        half = _R_S // NC
        nloop = (half + NS - 1) // NS

        def body(j, carry):
            rl = s + j * NS

            @pl.when(rl < half)
            def _():
                row = c * half + rl
                pltpu.sync_copy(d_hbm.at[row], idx_v)
                pltpu.sync_copy(buf_v, acc.at[idx_v], add=True)

            return carry

        lax.fori_loop(0, nloop, body, 0)
        plsc.subcore_barrier()
        rows = 1000

        @pl.when(s < N // rows)
        def _():
            pltpu.sync_copy(acc.at[pl.ds(s * rows, rows)],
                            p_hbm.at[c, pl.ds(s * rows, rows)])

    return _sc_gather, _sc_scatter, _sc_count


def _gather_rows(table, j2):
    return _sc_kernels()[0](table, j2)


def _scatter_rows(x, d2, zeros_nh):
    return _sc_kernels()[1](x, d2, zeros_nh)


def _count_rows(d2, ones16, zeros16):
    return _sc_kernels()[2](d2, ones16, zeros16)


def _lin_w(p, name):
    return p[name]["w"], p[name]["b"].reshape(1, H)


def kernel(node_features, edge_features, edge_index, params):
    src = edge_index[0]
    dst = edge_index[1]
    pad = _EP - 2 * E
    j2 = jnp.concatenate(
        [src, dst + N, jnp.zeros((pad,), jnp.int32)]).reshape(_R_GP, CH)
    d2 = dst.reshape(_R_S, CH)

    zeros_nh = jnp.zeros((N, H), _f32)
    zeros16 = jnp.zeros((N, H), _f32)
    ones16 = jnp.ones((CH, H), _f32)

    ne = params["node_enc"]
    ee = params["edge_enc"]
    procs = params["procs"]
    dec = params["dec"]

    def mlp_args(p):
        w1, b1 = _lin_w(p, "l1")
        w2, b2 = _lin_w(p, "l2")
        w3, b3 = _lin_w(p, "l3")
        return (w1, b1, w2, b2, w3, b3,
                p["gamma"].reshape(1, H), p["beta"].reshape(1, H))

    ew = []
    for p in procs:
        w1 = p["edge_mlp"]["l1"]["w"]
        ew.append((w1[:H], w1[H:2 * H], w1[2 * H:]))

    enc_n_args = mlp_args(ne)
    n0, ab = pl.pallas_call(
        _enc_node_body,
        grid=(N // BN,),
        in_specs=[pl.BlockSpec((BN, ND), lambda i: (i, 0)),
                  _w((ND, H)), _w((1, H)), _w((H, H)), _w((1, H)),
                  _w((H, H)), _w((1, H)), _w((1, H)), _w((1, H)),
                  _w((H, H)), _w((H, H))],
        out_specs=[pl.BlockSpec((BN, H), lambda i: (i, 0)),
                   pl.BlockSpec((2, BN, H), lambda i: (0, i, 0))],
        out_shape=[jax.ShapeDtypeStruct((N, H), _f32),
                   jax.ShapeDtypeStruct((2, N, H), _f32)],
    )(node_features, *enc_n_args, ew[0][0], ew[0][1])

    enc_e_args = mlp_args(ee)
    e = pl.pallas_call(
        _enc_edge_body,
        grid=(E // BE,),
        in_specs=[pl.BlockSpec((BE, ED), lambda i: (i, 0)),
                  _w((ED, H)), _w((1, H)), _w((H, H)), _w((1, H)),
                  _w((H, H)), _w((1, H)), _w((1, H)), _w((1, H))],
        out_specs=pl.BlockSpec((BE, H), lambda i: (i, 0)),
        out_shape=jax.ShapeDtypeStruct((E, H), _f32),
    )(edge_features, *enc_e_args)

    pc = _count_rows(d2, ones16, zeros16)

    n = n0
    for step in range(4):
        p = procs[step]
        table = ab.reshape(2 * N, H)
        g = _gather_rows(table, j2)

        em = mlp_args(p["edge_mlp"])
        w1c = ew[step][2]
        e = pl.pallas_call(
            _edge_body,
            grid=(E // BE,),
            in_specs=[pl.BlockSpec((BE, H), lambda i: (i, 0)),
                      pl.BlockSpec((BE, H), lambda i: (E // BE + i, 0)),
                      pl.BlockSpec((BE, H), lambda i: (i, 0)),
                      _w((H, H)), _w((1, H)), _w((H, H)), _w((1, H)),
                      _w((H, H)), _w((1, H)), _w((1, H)), _w((1, H))],
            out_specs=pl.BlockSpec((BE, H), lambda i: (i, 0)),
            out_shape=jax.ShapeDtypeStruct((E, H), _f32),
        )(g, g, e, w1c, em[1], em[2], em[3], em[4], em[5], em[6], em[7])

        pe = _scatter_rows(e, d2, zeros_nh)

        nm = mlp_args(p["node_mlp"])
        w1full = p["node_mlp"]["l1"]["w"]
        w1n, w1a = w1full[:H], w1full[H:]
        common_specs = [
            pl.BlockSpec((BN, H), lambda i: (i, 0)),
            pl.BlockSpec((2, BN, H), lambda i: (0, i, 0)),
            pl.BlockSpec((2, BN, H), lambda i: (0, i, 0)),
            _w((H, H)), _w((H, H)), _w((1, H)), _w((H, H)), _w((1, H)),
            _w((H, H)), _w((1, H)), _w((1, H)), _w((1, H))]
        if step < 3:
            n, ab = pl.pallas_call(
                _node_body,
                grid=(N // BN,),
                in_specs=common_specs + [_w((H, H)), _w((H, H))],
                out_specs=[pl.BlockSpec((BN, H), lambda i: (i, 0)),
                           pl.BlockSpec((2, BN, H), lambda i: (0, i, 0))],
                out_shape=[jax.ShapeDtypeStruct((N, H), _f32),
                           jax.ShapeDtypeStruct((2, N, H), _f32)],
            )(n, pe, pc, w1n, w1a, nm[1], nm[2], nm[3], nm[4], nm[5],
              nm[6], nm[7], ew[step + 1][0], ew[step + 1][1])
        else:
            wd1, bd1 = _lin_w(dec, "l1")
            wd2, bd2 = _lin_w(dec, "l2")
            wd3 = jnp.pad(dec["l3"]["w"], ((0, 0), (0, H - 3)))
            bd3 = jnp.pad(dec["l3"]["b"], (0, H - 3)).reshape(1, H)
            out = pl.pallas_call(
                _node_final_body,
                grid=(N // BN,),
                in_specs=common_specs + [_w((H, H)), _w((1, H)),
                                         _w((H, H)), _w((1, H)),
                                         _w((H, H)), _w((1, H))],
                out_specs=pl.BlockSpec((BN, H), lambda i: (i, 0)),
                out_shape=jax.ShapeDtypeStruct((N, H), _f32),
            )(n, pe, pc, w1n, w1a, nm[1], nm[2], nm[3], nm[4], nm[5],
              nm[6], nm[7], wd1, bd1, wd2, bd2, wd3, bd3)

    return out[:, :3]


# speedup vs baseline: 1.6933x; 1.1250x over previous
import functools

import jax
import jax.numpy as jnp
from jax import lax
from jax.experimental import pallas as pl
from jax.experimental.pallas import tpu as pltpu
from jax.experimental.pallas import tpu_sc as plsc

N = 10000
E = 160000
H = 128
ND = 10
ED = 9

NC = 2
NS = 16
NW = NC * NS
CH = 128

BN = 1000
BE = 1000

_f32 = jnp.float32


def _ln(y, g, b):
    mu = jnp.mean(y, axis=-1, keepdims=True)
    var = jnp.mean((y - mu) ** 2, axis=-1, keepdims=True)
    return (y - mu) / jnp.sqrt(var + 1e-5) * g + b


def _w(shape):
    return pl.BlockSpec(shape, lambda i: (0,) * len(shape))


def _enc_node_body(x, w1, b1, w2, b2, w3, b3, g, bt, wi, wj, n_out, ab_out):
    h = jnp.maximum(x[...] @ w1[...] + b1[...], 0.0)
    h = jnp.maximum(h @ w2[...] + b2[...], 0.0)
    y = h @ w3[...] + b3[...]
    n = _ln(y, g[...], bt[...])
    n_out[...] = n
    ab_out[0] = n @ wi[...]
    ab_out[1] = n @ wj[...]


def _enc_edge_body(x, w1, b1, w2, b2, w3, b3, g, bt, e_out):
    h = jnp.maximum(x[...] @ w1[...] + b1[...], 0.0)
    h = jnp.maximum(h @ w2[...] + b2[...], 0.0)
    y = h @ w3[...] + b3[...]
    e_out[...] = _ln(y, g[...], bt[...])


def _edge_body(ga, gb, e, w1c, b1, w2, b2, w3, b3, g, bt, out):
    ev = e[...]
    t = jnp.maximum(ga[...] + gb[...] + ev @ w1c[...] + b1[...], 0.0)
    h = jnp.maximum(t @ w2[...] + b2[...], 0.0)
    y = h @ w3[...] + b3[...]
    out[...] = _ln(y, g[...], bt[...]) + ev


def _node_body(nref, pe, pc, w1n, w1a, b1, w2, b2, w3, b3, g, bt, wi, wj,
               n_out, ab_out):
    n = nref[...]
    cnt = jnp.maximum((pc[0] + pc[1])[:, 0:1], 1.0)
    agg = (pe[0] + pe[1]) / cnt
    x = jnp.maximum(n @ w1n[...] + agg @ w1a[...] + b1[...], 0.0)
    h = jnp.maximum(x @ w2[...] + b2[...], 0.0)
    y = h @ w3[...] + b3[...]
    nn = _ln(y, g[...], bt[...]) + n
    n_out[...] = nn
    ab_out[0] = nn @ wi[...]
    ab_out[1] = nn @ wj[...]


def _node_final_body(nref, pe, pc, w1n, w1a, b1, w2, b2, w3, b3, g, bt,
                     wd1, bd1, wd2, bd2, wd3, bd3, out):
    n = nref[...]
    cnt = jnp.maximum((pc[0] + pc[1])[:, 0:1], 1.0)
    agg = (pe[0] + pe[1]) / cnt
    x = jnp.maximum(n @ w1n[...] + agg @ w1a[...] + b1[...], 0.0)
    h = jnp.maximum(x @ w2[...] + b2[...], 0.0)
    y = h @ w3[...] + b3[...]
    nn = _ln(y, g[...], bt[...]) + n
    d = jnp.maximum(nn @ wd1[...] + bd1[...], 0.0)
    d = jnp.maximum(d @ wd2[...] + bd2[...], 0.0)
    out[...] = d @ wd3[...] + bd3[...]


_R_S = E // CH
_NBUF = 4
_NRT = 80
_R_GP = _NRT * NW
_EP = _R_GP * CH


@functools.lru_cache(maxsize=None)
def _sc_kernels():
    mesh = plsc.VectorSubcoreMesh(core_axis_name="c", subcore_axis_name="s",
                                  num_cores=NC, num_subcores=NS)

    @functools.partial(
        pl.kernel,
        out_type=jax.ShapeDtypeStruct((2 * E, H), _f32),
        mesh=mesh,
        scratch_types=[
            pltpu.VMEM((_NBUF, CH), jnp.int32),
            pltpu.VMEM((_NBUF, CH, H), _f32),
            [pltpu.SemaphoreType.DMA] * _NBUF,
            [pltpu.SemaphoreType.DMA] * _NBUF,
        ],
    )
    def _sc_gather(t_hbm, j_hbm, g_hbm, idx_v, buf_v, gsem, wsem):
        w = lax.axis_index("s") * NC + lax.axis_index("c")
        nrow = 2 * E // CH

        def row(j):
            return w + j * NW

        def fetch_issue(j, slot):
            pltpu.sync_copy(j_hbm.at[row(j)], idx_v.at[slot])
            pltpu.async_copy(t_hbm.at[idx_v.at[slot]], buf_v.at[slot],
                             gsem[slot])

        def wait_gather(slot):
            pltpu.make_async_copy(t_hbm.at[idx_v.at[slot]], buf_v.at[slot],
                                  gsem[slot]).wait()

        def wait_wb(slot):
            pltpu.make_async_copy(g_hbm.at[pl.ds(0, CH)], buf_v.at[slot],
                                  wsem[slot]).wait()

        fetch_issue(0, 0)
        fetch_issue(1, 1)

        def outer(k, carry):
            for b in range(_NBUF):
                j = _NBUF * k + b
                nxt = j + 2
                nb = (b + 2) % _NBUF

                @pl.when(row(nxt) < nrow)
                def _():
                    @pl.when(nxt >= _NBUF)
                    def _():
                        wait_wb(nb)
                    fetch_issue(nxt, nb)

                @pl.when(row(j) < nrow)
                def _():
                    wait_gather(b)
                    pltpu.async_copy(buf_v.at[b],
                                     g_hbm.at[pl.ds(row(j) * CH, CH)],
                                     wsem[b])
            return carry

        lax.fori_loop(0, _NRT // _NBUF, outer, 0)
        for b in range(_NBUF):
            wait_wb(b)

    @functools.partial(
        pl.kernel,
        out_type=jax.ShapeDtypeStruct((NC, N, H), _f32),
        mesh=mesh,
        scratch_types=[
            pltpu.VMEM((CH,), jnp.int32),
            pltpu.VMEM((CH, H), _f32),
            pltpu.VMEM_SHARED((N, H), _f32),
            pltpu.SemaphoreType.DMA,
        ],
    )
    def _sc_scatter(x_hbm, d_hbm, z_hbm, p_hbm, idx_v, buf_v, acc, sem):
        c = lax.axis_index("c")
        s = lax.axis_index("s")

        @pl.when(s == 0)
        def _():
            pltpu.sync_copy(z_hbm, acc)

        plsc.subcore_barrier()

        half = _R_S // NC
        nloop = (half + NS - 1) // NS

        def body(j, carry):
            rl = s + j * NS

            @pl.when(rl < half)
            def _():
                row = c * half + rl
                pltpu.sync_copy(d_hbm.at[row], idx_v)
                pltpu.sync_copy(x_hbm.at[pl.ds(row * CH, CH)], buf_v)
                pltpu.sync_copy(buf_v, acc.at[idx_v], add=True)

            return carry

        lax.fori_loop(0, nloop, body, 0)
        plsc.subcore_barrier()
        rows = 1000

        @pl.when(s < N // rows)
        def _():
            pltpu.sync_copy(acc.at[pl.ds(s * rows, rows)],
                            p_hbm.at[c, pl.ds(s * rows, rows)])

    @functools.partial(
        pl.kernel,
        out_type=jax.ShapeDtypeStruct((NC, N, H), _f32),
        mesh=mesh,
        scratch_types=[
            pltpu.VMEM((CH,), jnp.int32),
            pltpu.VMEM((CH, H), _f32),
            pltpu.VMEM_SHARED((N, H), _f32),
            pltpu.SemaphoreType.DMA,
        ],
    )
    def _sc_count(d_hbm, ones_hbm, z_hbm, p_hbm, idx_v, buf_v, acc, sem):
        c = lax.axis_index("c")
        s = lax.axis_index("s")

        @pl.when(s == 0)
        def _():
            pltpu.sync_copy(z_hbm, acc)

        pltpu.sync_copy(ones_hbm, buf_v)
        plsc.subcore_barrier()

        half = _R_S // NC
        nloop = (half + NS - 1) // NS

        def body(j, carry):
            rl = s + j * NS

            @pl.when(rl < half)
            def _():
                row = c * half + rl
                pltpu.sync_copy(d_hbm.at[row], idx_v)
                pltpu.sync_copy(buf_v, acc.at[idx_v], add=True)

            return carry

        lax.fori_loop(0, nloop, body, 0)
        plsc.subcore_barrier()
        rows = 1000

        @pl.when(s < N // rows)
        def _():
            pltpu.sync_copy(acc.at[pl.ds(s * rows, rows)],
                            p_hbm.at[c, pl.ds(s * rows, rows)])

    return _sc_gather, _sc_scatter, _sc_count


def _gather_rows(table, j2):
    return _sc_kernels()[0](table, j2)


def _scatter_rows(x, d2, zeros_nh):
    return _sc_kernels()[1](x, d2, zeros_nh)


def _count_rows(d2, ones16, zeros16):
    return _sc_kernels()[2](d2, ones16, zeros16)


def _lin_w(p, name):
    return p[name]["w"], p[name]["b"].reshape(1, H)


def kernel(node_features, edge_features, edge_index, params):
    src = edge_index[0]
    dst = edge_index[1]
    pad = _EP - 2 * E
    j2 = jnp.concatenate(
        [src, dst + N, jnp.zeros((pad,), jnp.int32)]).reshape(_R_GP, CH)
    d2 = dst.reshape(_R_S, CH)

    zeros_nh = jnp.zeros((N, H), _f32)
    zeros16 = jnp.zeros((N, H), _f32)
    ones16 = jnp.ones((CH, H), _f32)

    ne = params["node_enc"]
    ee = params["edge_enc"]
    procs = params["procs"]
    dec = params["dec"]

    def mlp_args(p):
        w1, b1 = _lin_w(p, "l1")
        w2, b2 = _lin_w(p, "l2")
        w3, b3 = _lin_w(p, "l3")
        return (w1, b1, w2, b2, w3, b3,
                p["gamma"].reshape(1, H), p["beta"].reshape(1, H))

    ew = []
    for p in procs:
        w1 = p["edge_mlp"]["l1"]["w"]
        ew.append((w1[:H], w1[H:2 * H], w1[2 * H:]))

    enc_n_args = mlp_args(ne)
    n0, ab = pl.pallas_call(
        _enc_node_body,
        grid=(N // BN,),
        in_specs=[pl.BlockSpec((BN, ND), lambda i: (i, 0)),
                  _w((ND, H)), _w((1, H)), _w((H, H)), _w((1, H)),
                  _w((H, H)), _w((1, H)), _w((1, H)), _w((1, H)),
                  _w((H, H)), _w((H, H))],
        out_specs=[pl.BlockSpec((BN, H), lambda i: (i, 0)),
                   pl.BlockSpec((2, BN, H), lambda i: (0, i, 0))],
        out_shape=[jax.ShapeDtypeStruct((N, H), _f32),
                   jax.ShapeDtypeStruct((2, N, H), _f32)],
    )(node_features, *enc_n_args, ew[0][0], ew[0][1])

    enc_e_args = mlp_args(ee)
    e = pl.pallas_call(
        _enc_edge_body,
        grid=(E // BE,),
        in_specs=[pl.BlockSpec((BE, ED), lambda i: (i, 0)),
                  _w((ED, H)), _w((1, H)), _w((H, H)), _w((1, H)),
                  _w((H, H)), _w((1, H)), _w((1, H)), _w((1, H))],
        out_specs=pl.BlockSpec((BE, H), lambda i: (i, 0)),
        out_shape=jax.ShapeDtypeStruct((E, H), _f32),
    )(edge_features, *enc_e_args)

    pc = _count_rows(d2, ones16, zeros16)

    n = n0
    for step in range(4):
        p = procs[step]
        table = ab.reshape(2 * N, H)
        g = _gather_rows(table, j2)

        em = mlp_args(p["edge_mlp"])
        w1c = ew[step][2]
        e = pl.pallas_call(
            _edge_body,
            grid=(E // BE,),
            in_specs=[pl.BlockSpec((BE, H), lambda i: (i, 0)),
                      pl.BlockSpec((BE, H), lambda i: (E // BE + i, 0)),
                      pl.BlockSpec((BE, H), lambda i: (i, 0)),
                      _w((H, H)), _w((1, H)), _w((H, H)), _w((1, H)),
                      _w((H, H)), _w((1, H)), _w((1, H)), _w((1, H))],
            out_specs=pl.BlockSpec((BE, H), lambda i: (i, 0)),
            out_shape=jax.ShapeDtypeStruct((E, H), _f32),
        )(g, g, e, w1c, em[1], em[2], em[3], em[4], em[5], em[6], em[7])

        pe = _scatter_rows(e, d2, zeros_nh)

        nm = mlp_args(p["node_mlp"])
        w1full = p["node_mlp"]["l1"]["w"]
        w1n, w1a = w1full[:H], w1full[H:]
        common_specs = [
            pl.BlockSpec((BN, H), lambda i: (i, 0)),
            pl.BlockSpec((2, BN, H), lambda i: (0, i, 0)),
            pl.BlockSpec((2, BN, H), lambda i: (0, i, 0)),
            _w((H, H)), _w((H, H)), _w((1, H)), _w((H, H)), _w((1, H)),
            _w((H, H)), _w((1, H)), _w((1, H)), _w((1, H))]
        if step < 3:
            n, ab = pl.pallas_call(
                _node_body,
                grid=(N // BN,),
                in_specs=common_specs + [_w((H, H)), _w((H, H))],
                out_specs=[pl.BlockSpec((BN, H), lambda i: (i, 0)),
                           pl.BlockSpec((2, BN, H), lambda i: (0, i, 0))],
                out_shape=[jax.ShapeDtypeStruct((N, H), _f32),
                           jax.ShapeDtypeStruct((2, N, H), _f32)],
            )(n, pe, pc, w1n, w1a, nm[1], nm[2], nm[3], nm[4], nm[5],
              nm[6], nm[7], ew[step + 1][0], ew[step + 1][1])
        else:
            wd1, bd1 = _lin_w(dec, "l1")
            wd2, bd2 = _lin_w(dec, "l2")
            wd3 = jnp.pad(dec["l3"]["w"], ((0, 0), (0, H - 3)))
            bd3 = jnp.pad(dec["l3"]["b"], (0, H - 3)).reshape(1, H)
            out = pl.pallas_call(
                _node_final_body,
                grid=(N // BN,),
                in_specs=common_specs + [_w((H, H)), _w((1, H)),
                                         _w((H, H)), _w((1, H)),
                                         _w((H, H)), _w((1, H))],
                out_specs=pl.BlockSpec((BN, H), lambda i: (i, 0)),
                out_shape=jax.ShapeDtypeStruct((N, H), _f32),
            )(n, pe, pc, w1n, w1a, nm[1], nm[2], nm[3], nm[4], nm[5],
              nm[6], nm[7], wd1, bd1, wd2, bd2, wd3, bd3)

    return out[:, :3]
